# baseline (speedup 1.0000x reference)
import functools
import math

import numpy as np
import jax
import jax.numpy as jnp
from jax import lax
from jax.experimental import pallas as pl
from jax.experimental.pallas import tpu as pltpu
from jax.experimental.pallas import tpu_sc as plsc

D_MODEL = 1024
MAX_LENGTH = 8192
_PAD = 0
_B = 4
_NW = 32
_LW = MAX_LENGTH // _NW
_RS = 32
_NSUB = _LW // _RS


def _pe_table():
    position = np.arange(MAX_LENGTH, dtype=np.float64)[:, None]
    scale = -math.log(10000.0) / D_MODEL
    div = np.exp(np.arange(0, D_MODEL, 2, dtype=np.float64) * scale)
    pe = np.zeros((MAX_LENGTH, D_MODEL), dtype=np.float64)
    pe[:, 0::2] = np.sin(position * div)
    pe[:, 1::2] = np.cos(position * div)
    return pe.astype(np.float32)


_PE = _pe_table()

_mesh = plsc.VectorSubcoreMesh(core_axis_name="c", subcore_axis_name="s")


@functools.partial(
    pl.kernel,
    mesh=_mesh,
    out_type=jax.ShapeDtypeStruct((_B, MAX_LENGTH, D_MODEL), jnp.float32),
    scratch_types=[
        pltpu.VMEM((_RS, D_MODEL), jnp.float32),
        pltpu.VMEM((_RS, D_MODEL), jnp.float32),
        pltpu.VMEM((_RS, D_MODEL), jnp.float32),
        pltpu.VMEM((_RS, 16), jnp.float32),
    ],
)
def _sc_kernel(emb_hbm, msk_hbm, pe_hbm, out_hbm, pe_v, emb_v, out_v, msk_v):
    cid = lax.axis_index("c")
    sid = lax.axis_index("s")
    wid = sid * 2 + cid
    l0 = wid * _LW

    def sub_body(j, _):
        lbase = l0 + j * _RS
        pltpu.sync_copy(pe_hbm.at[pl.ds(lbase, _RS)], pe_v)

        def b_body(b, _):
            pltpu.sync_copy(emb_hbm.at[b, pl.ds(lbase, _RS)], emb_v)
            pltpu.sync_copy(msk_hbm.at[b, pl.ds(lbase, _RS)], msk_v)

            def r_body(row, _):
                m = msk_v[row, :]

                def k_body(k, _):
                    sl = pl.ds(k * 16, 16)
                    out_v[row, sl] = emb_v[row, sl] + pe_v[row, sl] * m
                    return 0

                lax.fori_loop(0, D_MODEL // 16, k_body, 0)
                return 0

            lax.fori_loop(0, _RS, r_body, 0)
            pltpu.sync_copy(out_v, out_hbm.at[b, pl.ds(lbase, _RS)])
            return 0

        lax.fori_loop(0, _B, b_body, 0)
        return 0

    lax.fori_loop(0, _NSUB, sub_body, 0)


def kernel(embedded, symbol):
    mask = jnp.broadcast_to(
        (symbol != _PAD).astype(jnp.float32)[:, :, None], symbol.shape + (16,))
    return _sc_kernel(embedded, mask, jnp.asarray(_PE))


# speedup vs baseline: 4.7250x; 4.7250x over previous
import math

import numpy as np
import jax
import jax.numpy as jnp
from jax.experimental import pallas as pl
from jax.experimental.pallas import tpu as pltpu

D_MODEL = 1024
MAX_LENGTH = 8192
_PAD = 0
_LB = 512
_N0 = 8
_NDBL = (_LB // _N0).bit_length() - 1


def _constants():
    scale = -math.log(10000.0) / D_MODEL
    theta = np.exp(np.arange(0, D_MODEL, 2, dtype=np.float64) * scale)
    pos = np.arange(_N0, dtype=np.float64)[:, None]
    init = np.zeros((2, _N0, D_MODEL), dtype=np.float64)
    init[0, :, 0::2] = np.sin(pos * theta)
    init[0, :, 1::2] = np.cos(pos * theta)
    init[1, :, 0::2] = init[0, :, 1::2]
    init[1, :, 1::2] = init[0, :, 0::2]
    deltas = [_N0 << s for s in range(_NDBL)] + [_LB]
    rot = np.zeros((len(deltas), 2, D_MODEL), dtype=np.float64)
    for j, dlt in enumerate(deltas):
        rot[j, 0, :] = np.repeat(np.cos(dlt * theta), 2)
        s = np.repeat(np.sin(dlt * theta), 2)
        s[1::2] *= -1.0
        rot[j, 1, :] = s
    return init.astype(np.float32), rot.astype(np.float32)


_INIT, _ROT = _constants()


def _body(sym_ref, emb_ref, init_ref, rot_ref, out_ref, v_ref, w_ref):
    i = pl.program_id(0)
    nl = pl.num_programs(0)
    p = jax.lax.rem(i, 2)

    @pl.when(i == 0)
    def _init():
        v_ref[0, 0:_N0, :] = init_ref[0]
        w_ref[0, 0:_N0, :] = init_ref[1]
        for s in range(_NDBL):
            size = _N0 << s
            c = rot_ref[s, 0:1, :]
            sn = rot_ref[s, 1:2, :]
            v = v_ref[0, 0:size, :]
            w = w_ref[0, 0:size, :]
            v_ref[0, size:2 * size, :] = v * c + w * sn
            w_ref[0, size:2 * size, :] = w * c - v * sn

    @pl.when(i < nl - 1)
    def _rot_next():
        c = rot_ref[_NDBL, 0:1, :]
        sn = rot_ref[_NDBL, 1:2, :]
        v = v_ref[p]
        w = w_ref[p]
        v_ref[1 - p] = v * c + w * sn
        w_ref[1 - p] = w * c - v * sn

    v = v_ref[p]
    for k in range(4):
        mask = (sym_ref[k] != _PAD).astype(jnp.float32)
        out_ref[k] = emb_ref[k] + v * mask


def kernel(embedded, symbol):
    B, L = symbol.shape
    nl = L // _LB
    sym3 = symbol.reshape(B, L, 1)
    return pl.pallas_call(
        _body,
        grid=(nl,),
        in_specs=[
            pl.BlockSpec((B, _LB, 1), lambda i: (0, i, 0)),
            pl.BlockSpec((B, _LB, D_MODEL), lambda i: (0, i, 0)),
            pl.BlockSpec((2, _N0, D_MODEL), lambda i: (0, 0, 0)),
            pl.BlockSpec((_NDBL + 1, 2, D_MODEL), lambda i: (0, 0, 0)),
        ],
        out_specs=pl.BlockSpec((B, _LB, D_MODEL), lambda i: (0, i, 0)),
        out_shape=jax.ShapeDtypeStruct((B, L, D_MODEL), jnp.float32),
        scratch_shapes=[
            pltpu.VMEM((2, _LB, D_MODEL), jnp.float32),
            pltpu.VMEM((2, _LB, D_MODEL), jnp.float32),
        ],
    )(sym3, embedded, jnp.asarray(_INIT), jnp.asarray(_ROT))
